# baseline (speedup 1.0000x reference)
import functools

import jax
import jax.numpy as jnp
from jax import lax
from jax.experimental import pallas as pl
from jax.experimental.pallas import tpu as pltpu
from jax.experimental.pallas import tpu_sc as plsc

_B, _N, _P = 4096, 2048, 50
_MARGIN = 1.0
_NC, _NS, _L = 2, 16, 16
_NW = _NC * _NS
_ROWS_PER_W = _B // _NW
_R = 32
_CHUNKS = _ROWS_PER_W // _R
_GROUPS = (0, 16, 32, 34)


def _loss_kernel(scores_hbm, pos_hbm, neg_hbm, out_hbm,
                 pb0, pb1, nb0, nb1, fb0, fb1, vb0, vb1, outv,
                 sem0, sem1, gsem0, gsem1):
    wid = lax.axis_index("s") * _NC + lax.axis_index("c")
    row0 = wid * _ROWS_PER_W
    flat = scores_hbm

    pbufs = (pb0, pb1)
    nbufs = (nb0, nb1)
    fbufs = (fb0, fb1)
    vbufs = (vb0, vb1)
    sems = (sem0, sem1)
    gsems = (gsem0, gsem1)

    def start_idx(c, slot):
        base = row0 + c * _R
        return [
            pltpu.async_copy(pos_hbm.at[pl.ds(base, _R)], pbufs[slot],
                             sems[slot]),
            pltpu.async_copy(neg_hbm.at[pl.ds(base, _R)], nbufs[slot],
                             sems[slot]),
        ]

    def fill_flat(c, slot):
        pb, nb, fb = pbufs[slot], nbufs[slot], fbufs[slot]
        base = row0 + c * _R

        def row_body(r, carry):
            rb = (base + r) * _N
            for g, start in enumerate(_GROUPS):
                fb[r, pl.ds(g * _L, _L)] = pb[r, pl.ds(start, _L)] + rb
                fb[r, pl.ds(64 + g * _L, _L)] = nb[r, pl.ds(start, _L)] + rb
            return carry

        lax.fori_loop(0, _R, row_body, 0)

    def start_gather(slot):
        fb, vb = fbufs[slot], vbufs[slot]
        return [
            pltpu.async_copy(flat.at[fb.at[r]], vb.at[r], gsems[slot])
            for r in range(_R)
        ]

    lane = lax.iota(jnp.int32, _L)
    tail_new = lane >= (3 * _L - _GROUPS[3])
    zero16 = jnp.zeros((_L,), jnp.float32)
    izero16 = jnp.zeros((_L,), jnp.int32)
    cpos = [lane + g * _L for g in range(4)]
    cneg = [lane + 64 + g * _L for g in range(4)]

    def hinge(slot, acc):
        vb = vbufs[slot]

        def row_body(r, acc):
            rsplat = izero16 + r
            for g in range(4):
                ps = plsc.load_gather(vb, [rsplat, cpos[g]])
                ns = plsc.load_gather(vb, [rsplat, cneg[g]])
                loss = jnp.maximum(_MARGIN + ns - ps, 0.0)
                if g == 3:
                    loss = jnp.where(tail_new, loss, zero16)
                acc = acc + loss
            return acc

        return lax.fori_loop(0, _R, row_body, acc)

    acc = zero16
    pending_idx = start_idx(0, 0)
    pending_gather = []
    for c in range(_CHUNKS):
        slot = c % 2
        for h in pending_idx:
            h.wait()
        if c + 1 < _CHUNKS:
            pending_idx = start_idx(c + 1, (c + 1) % 2)
        fill_flat(c, slot)
        new_gather = start_gather(slot)
        if c > 0:
            for h in pending_gather:
                h.wait()
            acc = hinge((c - 1) % 2, acc)
        pending_gather = new_gather
    for h in pending_gather:
        h.wait()
    acc = hinge((_CHUNKS - 1) % 2, acc)

    outv[...] = acc * (1.0 / (_B * _P))
    pltpu.sync_copy(outv, out_hbm.at[wid])


@jax.jit
def kernel(saliency_scores, pos_indices, neg_indices):
    mesh = plsc.VectorSubcoreMesh(core_axis_name="c", subcore_axis_name="s")
    run = functools.partial(
        pl.kernel,
        out_type=jax.ShapeDtypeStruct((_NW, _L), jnp.float32),
        mesh=mesh,
        compiler_params=pltpu.CompilerParams(needs_layout_passes=False),
        scratch_types=[
            pltpu.VMEM((_R, _P), jnp.int32),
            pltpu.VMEM((_R, _P), jnp.int32),
            pltpu.VMEM((_R, _P), jnp.int32),
            pltpu.VMEM((_R, _P), jnp.int32),
            pltpu.VMEM((_R, 8 * _L), jnp.int32),
            pltpu.VMEM((_R, 8 * _L), jnp.int32),
            pltpu.VMEM((_R, 8 * _L), jnp.float32),
            pltpu.VMEM((_R, 8 * _L), jnp.float32),
            pltpu.VMEM((_L,), jnp.float32),
            pltpu.SemaphoreType.DMA,
            pltpu.SemaphoreType.DMA,
            pltpu.SemaphoreType.DMA,
            pltpu.SemaphoreType.DMA,
        ],
    )(_loss_kernel)
    scores_flat = jnp.reshape(saliency_scores, (_B * _N,))
    partials = run(scores_flat, pos_indices, neg_indices)
    return jnp.sum(partials)


# speedup vs baseline: 1.9251x; 1.9251x over previous
import functools

import jax
import jax.numpy as jnp
from jax import lax
from jax.experimental import pallas as pl
from jax.experimental.pallas import tpu as pltpu
from jax.experimental.pallas import tpu_sc as plsc

_B, _N, _P = 4096, 2048, 50
_MARGIN = 1.0
_NC, _NS, _L = 2, 16, 16
_NW = _NC * _NS
_ROWS_PER_W = _B // _NW
_R = 16
_CHUNKS = _ROWS_PER_W // _R
_NB = 3
_GROUPS = (0, 16, 32, 34)


def _loss_kernel(scores_hbm, idx_hbm, out_hbm,
                 sc0, sc1, sc2, ib0, ib1, ib2, outv,
                 sem0, sem1, sem2):
    wid = lax.axis_index("s") * _NC + lax.axis_index("c")
    row0 = wid * _ROWS_PER_W

    sbufs = (sc0, sc1, sc2)
    ibufs = (ib0, ib1, ib2)
    sems = (sem0, sem1, sem2)

    def start_chunk(c, slot):
        base = row0 + c * _R
        return [
            pltpu.async_copy(scores_hbm.at[pl.ds(base, _R)], sbufs[slot],
                             sems[slot]),
            pltpu.async_copy(idx_hbm.at[pl.ds(base, _R)], ibufs[slot],
                             sems[slot]),
        ]

    lane = lax.iota(jnp.int32, _L)
    tail_new = lane >= (3 * _L - _GROUPS[3])
    zero16 = jnp.zeros((_L,), jnp.float32)

    def make_row_body(sbuf, ibuf):
        def row_body(r, acc):
            rv = jnp.full((_L,), 0, jnp.int32) + r
            for g, start in enumerate(_GROUPS):
                pi = ibuf[r, pl.ds(start, _L)]
                ni = ibuf[r, pl.ds(_P + start, _L)]
                ps = plsc.load_gather(sbuf, [rv, pi])
                ns = plsc.load_gather(sbuf, [rv, ni])
                loss = jnp.maximum(_MARGIN + ns - ps, 0.0)
                if g == 3:
                    loss = jnp.where(tail_new, loss, zero16)
                acc = acc + loss
            return acc
        return row_body

    pending = [start_chunk(c, c) for c in range(_NB - 1)]
    pending.append([])
    acc = zero16
    for c in range(_CHUNKS):
        slot = c % _NB
        for h in pending[slot]:
            h.wait()
        if c + _NB - 1 < _CHUNKS:
            pending[(c + _NB - 1) % _NB] = start_chunk(
                c + _NB - 1, (c + _NB - 1) % _NB)
        acc = lax.fori_loop(
            0, _R, make_row_body(sbufs[slot], ibufs[slot]), acc)

    outv[...] = acc * (1.0 / (_B * _P))
    pltpu.sync_copy(outv, out_hbm.at[wid])


@jax.jit
def kernel(saliency_scores, pos_indices, neg_indices):
    mesh = plsc.VectorSubcoreMesh(core_axis_name="c", subcore_axis_name="s")
    run = functools.partial(
        pl.kernel,
        out_type=jax.ShapeDtypeStruct((_NW, _L), jnp.float32),
        mesh=mesh,
        compiler_params=pltpu.CompilerParams(needs_layout_passes=False),
        scratch_types=[
            pltpu.VMEM((_R, _N), jnp.float32),
            pltpu.VMEM((_R, _N), jnp.float32),
            pltpu.VMEM((_R, _N), jnp.float32),
            pltpu.VMEM((_R, 2 * _P), jnp.int32),
            pltpu.VMEM((_R, 2 * _P), jnp.int32),
            pltpu.VMEM((_R, 2 * _P), jnp.int32),
            pltpu.VMEM((_L,), jnp.float32),
            pltpu.SemaphoreType.DMA,
            pltpu.SemaphoreType.DMA,
            pltpu.SemaphoreType.DMA,
        ],
    )(_loss_kernel)
    idx = jnp.concatenate([pos_indices, neg_indices], axis=1)
    partials = run(saliency_scores, idx)
    return jnp.sum(partials)
